# baseline (speedup 1.0000x reference)
import functools

import jax
import jax.numpy as jnp
from jax import lax
from jax.experimental import pallas as pl
from jax.experimental.pallas import tpu as pltpu
from jax.experimental.pallas import tpu_sc as plsc

BATCH = 16384
EMB = 16
NC = 2
NS = 16
L = 16
NW = NC * NS
BPW = BATCH // NW
CH = 128
NCH = BPW // CH
GROUP = 8

_mesh = plsc.VectorSubcoreMesh(core_axis_name="c", subcore_axis_name="s")


@functools.partial(
    pl.kernel,
    out_type=jax.ShapeDtypeStruct((BATCH,), jnp.float32),
    mesh=_mesh,
    scratch_types=[
        pltpu.VMEM((NCH, CH), jnp.int32),
        pltpu.VMEM((NCH, CH), jnp.int32),
        pltpu.VMEM((NCH, CH), jnp.int32),
        pltpu.VMEM((NCH, CH), jnp.int32),
        pltpu.VMEM((CH, GROUP * EMB), jnp.float32),
        pltpu.VMEM((CH, GROUP * EMB), jnp.float32),
        pltpu.VMEM((CH, GROUP * EMB), jnp.float32),
        pltpu.VMEM((CH, GROUP * EMB), jnp.float32),
        pltpu.VMEM((BPW,), jnp.float32),
        pltpu.SemaphoreType.DMA,
        pltpu.SemaphoreType.DMA,
    ],
    compiler_params=pltpu.CompilerParams(needs_layout_passes=False),
)
def _mf_sc(u_id_hbm, i_id_hbm, user_hbm, item_hbm, out_hbm,
           u_raw, i_raw, u_gidx, i_gidx,
           u_buf0, u_buf1, i_buf0, i_buf1, out_v, sem0, sem1):
    wid = lax.axis_index("s") * NC + lax.axis_index("c")
    u_bufs = (u_buf0, u_buf1)
    i_bufs = (i_buf0, i_buf1)
    sems = (sem0, sem1)

    pltpu.sync_copy(u_id_hbm.at[pl.ds(wid * NCH, NCH)], u_raw)
    pltpu.sync_copy(i_id_hbm.at[pl.ds(wid * NCH, NCH)], i_raw)
    for c in range(NCH):
        for q in range(CH // L):
            sl = pl.ds(q * L, L)
            u_gidx[c, sl] = jnp.right_shift(u_raw[c, sl], 3)
            i_gidx[c, sl] = jnp.right_shift(i_raw[c, sl], 3)

    def start(c):
        nb = c % 2
        cu = pltpu.async_copy(user_hbm.at[u_gidx.at[c]], u_bufs[nb], sems[nb])
        ci = pltpu.async_copy(item_hbm.at[i_gidx.at[c]], i_bufs[nb], sems[nb])
        return cu, ci

    lanes = lax.iota(jnp.int32, L)
    inflight = start(0)
    for c in range(NCH):
        nb = c % 2
        cur = inflight
        if c + 1 < NCH:
            inflight = start(c + 1)
        cur[0].wait()
        cur[1].wait()
        for g in range(CH // L):
            sl = pl.ds(g * L, L)
            s_u = jnp.bitwise_and(u_raw[c, sl], 7) * EMB
            s_i = jnp.bitwise_and(i_raw[c, sl], 7) * EMB
            row = g * L + lanes
            acc = jnp.zeros((L,), jnp.float32)
            for e in range(EMB):
                uu = plsc.load_gather(u_bufs[nb], [row, s_u + e])
                ii = plsc.load_gather(i_bufs[nb], [row, s_i + e])
                acc = acc + uu * ii
            out_v[pl.ds(c * CH + g * L, L)] = acc

    pltpu.sync_copy(out_v, out_hbm.at[pl.ds(wid * BPW, BPW)])


def kernel(u_id, i_id, user_table, item_table):
    u2 = u_id.astype(jnp.int32).reshape(NW * NCH, CH)
    i2 = i_id.astype(jnp.int32).reshape(NW * NCH, CH)
    ut2 = user_table.reshape(user_table.shape[0] // GROUP, GROUP * EMB)
    it2 = item_table.reshape(item_table.shape[0] // GROUP, GROUP * EMB)
    return _mf_sc(u2, i2, ut2, it2)


# speedup vs baseline: 6.0858x; 6.0858x over previous
import functools

import jax
import jax.numpy as jnp
from jax import lax
from jax.experimental import pallas as pl
from jax.experimental.pallas import tpu as pltpu
from jax.experimental.pallas import tpu_sc as plsc

BATCH = 16384
EMB = 16
NC = 2
NS = 16
L = 16
NW = NC * NS
BPW = BATCH // NW
CH = 8
NCH = BPW // CH
BLK = 128
PAD = BPW + 2 * CH

_mesh = plsc.VectorSubcoreMesh(core_axis_name="c", subcore_axis_name="s")


@functools.partial(
    pl.kernel,
    out_type=jax.ShapeDtypeStruct((BATCH,), jnp.float32),
    mesh=_mesh,
    scratch_types=[
        pltpu.VMEM((PAD,), jnp.int32),
        pltpu.VMEM((PAD,), jnp.int32),
        pltpu.VMEM((CH, EMB, BLK), jnp.float32),
        pltpu.VMEM((CH, EMB, BLK), jnp.float32),
        pltpu.VMEM((CH, EMB, BLK), jnp.float32),
        pltpu.VMEM((CH, EMB, BLK), jnp.float32),
        pltpu.VMEM((PAD,), jnp.float32),
        pltpu.SemaphoreType.DMA,
        pltpu.SemaphoreType.DMA,
    ],
    compiler_params=pltpu.CompilerParams(
        needs_layout_passes=False, disable_bounds_checks=True),
)
def _mf_sc(u_id_hbm, i_id_hbm, user_hbm, item_hbm, out_hbm,
           u_vm, i_vm, u_buf0, u_buf1, i_buf0, i_buf1, out_v, sem0, sem1):
    wid = lax.axis_index("s") * NC + lax.axis_index("c")
    base = wid * BPW
    u_bufs = (u_buf0, u_buf1)
    i_bufs = (i_buf0, i_buf1)
    sems = (sem0, sem1)

    pltpu.sync_copy(u_id_hbm.at[pl.ds(base, BPW)], u_vm.at[pl.ds(0, BPW)])
    pltpu.sync_copy(i_id_hbm.at[pl.ds(base, BPW)], i_vm.at[pl.ds(0, BPW)])

    lanes = lax.iota(jnp.int32, L)

    def issue(c, nb):
        uv = u_vm[pl.ds(c * CH, L)]
        iv = i_vm[pl.ds(c * CH, L)]
        for j in range(CH):
            for (v, table, bufs) in ((uv, user_hbm, u_bufs),
                                     (iv, item_hbm, i_bufs)):
                blk = pl.multiple_of((v[j] >> 7) << 7, BLK)
                pltpu.async_copy(
                    table.at[pl.ds(0, EMB), pl.ds(blk, BLK)],
                    bufs[nb].at[j], sems[nb])

    def consume(c, nb):
        for j in range(CH):
            pltpu.make_async_copy(
                user_hbm.at[pl.ds(0, EMB), pl.ds(0, BLK)],
                u_bufs[nb].at[j], sems[nb]).wait()
            pltpu.make_async_copy(
                item_hbm.at[pl.ds(0, EMB), pl.ds(0, BLK)],
                i_bufs[nb].at[j], sems[nb]).wait()

        uv = u_vm[pl.ds(c * CH, L)]
        iv = i_vm[pl.ds(c * CH, L)]
        acc = jnp.zeros((L,), jnp.float32)
        for j in range(CH):
            col_u = jnp.broadcast_to(uv[j] & 127, (L,))
            col_i = jnp.broadcast_to(iv[j] & 127, (L,))
            slot = jnp.full((L,), j, jnp.int32)
            u_col = plsc.load_gather(u_bufs[nb], [slot, lanes, col_u])
            i_col = plsc.load_gather(i_bufs[nb], [slot, lanes, col_i])
            s = jnp.sum(u_col * i_col)
            acc = jnp.where(lanes == j, s, acc)
        plsc.store_compressed(out_v.at[pl.ds(c * CH, L)], acc, mask=lanes < CH)

    def body(p, carry):
        issue(2 * p + 1, 1)
        consume(2 * p, 0)

        @pl.when(p + 1 < NCH // 2)
        def _prefetch():
            issue(2 * p + 2, 0)

        consume(2 * p + 1, 1)
        return carry

    issue(0, 0)
    lax.fori_loop(0, NCH // 2, body, 0)

    pltpu.sync_copy(out_v.at[pl.ds(0, BPW)], out_hbm.at[pl.ds(base, BPW)])


def kernel(u_id, i_id, user_table, item_table):
    return _mf_sc(u_id.astype(jnp.int32), i_id.astype(jnp.int32),
                  user_table.T, item_table.T)
